# baseline (speedup 1.0000x reference)
import functools

import jax
import jax.numpy as jnp
from jax import lax
from jax.experimental import pallas as pl
from jax.experimental.pallas import tpu as pltpu
from jax.experimental.pallas import tpu_sc as plsc

N = 10000
E = 160000
H = 128
NH = 8
DH = H // NH
L = 3
NRBF = 50
MAXZ = 16
CUT_HI = 5.0

NB = 20
SEGB = 8192
NW = 32
CH = 5008
EPAD = 160256
EALLOC = 165120

_MESH = plsc.VectorSubcoreMesh(
    core_axis_name="c", subcore_axis_name="s", num_cores=2, num_subcores=16)


def _cv(ref, i):
    return ref[pl.ds(i, 16)][0]


def _silu(x):
    return x * jax.nn.sigmoid(x)


def _cosine_cutoff(d):
    return 0.5 * (jnp.cos(d * jnp.pi / CUT_HI) + 1.0) * (d < CUT_HI)


def _expnorm_rbf(d, means, betas):
    return jnp.exp(-betas[None, :] * (jnp.exp(-d[:, None]) - means[None, :]) ** 2)


def _prep_body(srce_hbm, dste_hbm, posf_hbm, z_hbm, d2_hbm, seg_hbm, cnt_hbm,
               posf_v, z_v, src_v, dst_v, d2_v, segv_v, cw_v, sm):
    c = lax.axis_index("c")
    s = lax.axis_index("s")
    w = s * 2 + c
    start = w * CH
    nume = jnp.where(w == NW - 1, E - (NW - 1) * CH, CH)
    nv = nume // 16

    pltpu.sync_copy(posf_hbm, posf_v)
    pltpu.sync_copy(z_hbm, z_v)
    pltpu.sync_copy(srce_hbm.at[pl.ds(start, CH)], src_v)
    pltpu.sync_copy(dste_hbm.at[pl.ds(start, CH)], dst_v)

    lanes = lax.iota(jnp.int32, 16)

    def vec_body(i, carry):
        cv0, cv1 = carry
        srcv = src_v[pl.ds(i * 16, 16)]
        dstv = dst_v[pl.ds(i * 16, 16)]
        s3 = srcv * 3
        t3 = dstv * 3
        dx = plsc.load_gather(posf_v, [s3]) - plsc.load_gather(posf_v, [t3])
        dy = plsc.load_gather(posf_v, [s3 + 1]) - plsc.load_gather(posf_v, [t3 + 1])
        dz = plsc.load_gather(posf_v, [s3 + 2]) - plsc.load_gather(posf_v, [t3 + 2])
        d2 = dx * dx + dy * dy + dz * dz
        zsv = plsc.load_gather(z_v, [srcv])
        d2_v[pl.ds(i * 16, 16)] = d2
        segv_v[pl.ds(i * 16, 16)] = dstv * MAXZ + zsv
        bv = lax.shift_right_logical(dstv, 9)
        for b in range(16):
            cv0 = cv0 + jnp.sum((bv == b).astype(jnp.int32)) * (lanes == b).astype(jnp.int32)
        for b in range(16, NB):
            cv1 = cv1 + jnp.sum((bv == b).astype(jnp.int32)) * (lanes == (b - 16)).astype(jnp.int32)
        return cv0, cv1

    zv16 = jnp.zeros((16,), jnp.int32)
    cv0, cv1 = lax.fori_loop(0, nv, vec_body, (zv16, zv16))
    pad8 = lambda v: lax.shift_left(lax.shift_right_logical(v + 7, 3), 3)
    cw_v[pl.ds(0, 16)] = pad8(cv0)
    cw_v[pl.ds(16, 16)] = jnp.where(lanes < NB - 16, pad8(cv1), 0)
    pltpu.sync_copy(cw_v, cnt_hbm.at[w])
    pltpu.sync_copy(d2_v, d2_hbm.at[pl.ds(start, CH)])
    pltpu.sync_copy(segv_v, seg_hbm.at[pl.ds(start, CH)])


@jax.jit
def _sc_prep(srcp, dstp, posf, z):
    return pl.kernel(
        _prep_body,
        out_type=(
            jax.ShapeDtypeStruct((EPAD,), jnp.float32),
            jax.ShapeDtypeStruct((EPAD,), jnp.int32),
            jax.ShapeDtypeStruct((NW, 32), jnp.int32),
        ),
        mesh=_MESH,
        scratch_types=[
            pltpu.VMEM((N * 3,), jnp.float32),
            pltpu.VMEM((N,), jnp.int32),
            pltpu.VMEM((CH,), jnp.int32),
            pltpu.VMEM((CH,), jnp.int32),
            pltpu.VMEM((CH,), jnp.float32),
            pltpu.VMEM((CH,), jnp.int32),
            pltpu.VMEM((32,), jnp.int32),
            pltpu.SMEM((32,), jnp.int32),
        ],
        compiler_params=pltpu.CompilerParams(needs_layout_passes=False),
    )(srcp, dstp, posf, z)


def _place_body(srce_hbm, dste_hbm, sege_hbm, cnts_hbm,
                srcs_hbm, segs_hbm, eids_hbm, cntp_hbm,
                src_v, dst_v, seg_v, bsrc_v, bseg_v, beid_v, cnts_v, cnt_loc,
                sm):
    c = lax.axis_index("c")
    s = lax.axis_index("s")
    w = s * 2 + c
    start = w * CH
    nume = jnp.where(w == NW - 1, E - (NW - 1) * CH, CH)
    nv = nume // 16
    lanes = lax.iota(jnp.int32, 16)

    pltpu.sync_copy(srce_hbm.at[pl.ds(start, CH)], src_v)
    pltpu.sync_copy(dste_hbm.at[pl.ds(start, CH)], dst_v)
    pltpu.sync_copy(sege_hbm.at[pl.ds(start, CH)], seg_v)
    pltpu.sync_copy(cnts_hbm, cnts_v.at[pl.ds(0, NW * 32)])

    loc = jnp.int32(0)
    gb = jnp.int32(0)
    for b in range(NB):
        sm[b] = loc
        sm[40 + b] = loc
        loc = loc + _cv(cnts_v, w * 32 + b)
        part = lax.fori_loop(0, w, lambda wp, a: a + _cv(cnts_v, wp * 32 + b), jnp.int32(0))
        tot = lax.fori_loop(0, NW, lambda wp, a: a + _cv(cnts_v, wp * 32 + b), jnp.int32(0))
        sm[20 + b] = gb + part
        gb = gb + tot

    def vec_body(i, _):
        srcv = src_v[pl.ds(i * 16, 16)]
        dstv = dst_v[pl.ds(i * 16, 16)]
        segv = seg_v[pl.ds(i * 16, 16)]
        eidv = start + i * 16 + lanes
        bv = lax.shift_right_logical(dstv, 9)
        for b in range(NB):
            m = bv == b
            off = sm[40 + b]
            plsc.store_compressed(bsrc_v.at[pl.ds(off, 16)], srcv, mask=m)
            plsc.store_compressed(bseg_v.at[pl.ds(off, 16)], segv, mask=m)
            plsc.store_compressed(beid_v.at[pl.ds(off, 16)], eidv, mask=m)
            sm[40 + b] = off + jnp.sum(m.astype(jnp.int32))
        return _

    lax.fori_loop(0, nv, vec_body, None)

    z16 = jnp.zeros((16,), jnp.int32)
    for b in range(NB):
        cur = sm[40 + b]
        end = sm[b] + _cv(cnts_v, w * 32 + b)
        m = lanes < (end - cur)
        plsc.store_compressed(bsrc_v.at[pl.ds(cur, 16)], z16, mask=m)
        plsc.store_compressed(bseg_v.at[pl.ds(cur, 16)],
                              jnp.full((16,), b * SEGB + SEGB, jnp.int32), mask=m)
        plsc.store_compressed(beid_v.at[pl.ds(cur, 16)], z16, mask=m)

    for b in range(NB):
        lo = sm[b]
        gbase = sm[20 + b]
        pcnt = _cv(cnts_v, w * 32 + b)
        n128 = lax.shift_right_logical(pcnt, 7)
        n8 = lax.shift_right_logical(pcnt - n128 * 128, 3)

        def big(j, _, lo=lo, gbase=gbase):
            go = pl.multiple_of(gbase + j * 128, 8)
            lo8 = pl.multiple_of(lo + j * 128, 8)
            pltpu.sync_copy(bsrc_v.at[pl.ds(lo8, 128)], srcs_hbm.at[pl.ds(go, 128)])
            pltpu.sync_copy(bseg_v.at[pl.ds(lo8, 128)], segs_hbm.at[pl.ds(go, 128)])
            pltpu.sync_copy(beid_v.at[pl.ds(lo8, 128)], eids_hbm.at[pl.ds(go, 128)])
            return _

        def small(j, _, lo=lo, gbase=gbase, n128=n128):
            o = n128 * 128 + j * 8
            go = pl.multiple_of(gbase + o, 8)
            lo8 = pl.multiple_of(lo + o, 8)
            pltpu.sync_copy(bsrc_v.at[pl.ds(lo8, 8)], srcs_hbm.at[pl.ds(go, 8)])
            pltpu.sync_copy(bseg_v.at[pl.ds(lo8, 8)], segs_hbm.at[pl.ds(go, 8)])
            pltpu.sync_copy(beid_v.at[pl.ds(lo8, 8)], eids_hbm.at[pl.ds(go, 8)])
            return _

        lax.fori_loop(0, n128, big, None)
        lax.fori_loop(0, n8, small, None)

    ones16 = jnp.ones((16,), jnp.int32)
    for b in range(NB):
        lo = sm[b]
        pcnt = _cv(cnts_v, w * 32 + b)

        def zero(j, _):
            cnt_loc[pl.ds(j * 16, 16)] = z16
            return _

        lax.fori_loop(0, 513, zero, None)

        def count(j, _, lo=lo, pcnt=pcnt):
            sv = bseg_v[pl.ds(lo + j * 16, 16)]
            idx = sv - b * SEGB
            m = lanes < (pcnt - j * 16)
            plsc.addupdate_scatter(cnt_loc, [idx], ones16, mask=m)
            return _

        lax.fori_loop(0, lax.shift_right_logical(pcnt + 15, 4), count, None)
        pltpu.sync_copy(cnt_loc, cntp_hbm.at[pl.ds(pl.multiple_of(w * (NB * 8208) + b * 8208, 8), 8208)])


@jax.jit
def _sc_place(srcp, dstp, segp, cnts):
    return pl.kernel(
        _place_body,
        out_type=(
            jax.ShapeDtypeStruct((EALLOC,), jnp.int32),
            jax.ShapeDtypeStruct((EALLOC,), jnp.int32),
            jax.ShapeDtypeStruct((EALLOC,), jnp.int32),
            jax.ShapeDtypeStruct((NW * NB * 8208,), jnp.int32),
        ),
        mesh=_MESH,
        scratch_types=[
            pltpu.VMEM((CH,), jnp.int32),
            pltpu.VMEM((CH,), jnp.int32),
            pltpu.VMEM((CH,), jnp.int32),
            pltpu.VMEM((5184,), jnp.int32),
            pltpu.VMEM((5184,), jnp.int32),
            pltpu.VMEM((5184,), jnp.int32),
            pltpu.VMEM((NW * 32 + 16,), jnp.int32),
            pltpu.VMEM((8208,), jnp.int32),
            pltpu.SMEM((64,), jnp.int32),
        ],
        compiler_params=pltpu.CompilerParams(needs_layout_passes=False),
    )(srcp, dstp, segp, cnts)


def _make_segsum(with_xc):
    def body(*refs):
        if with_xc:
            (wf_hbm, xc_hbm, srcs_hbm, segs_hbm, eids_hbm, cnts_hbm, y_hbm,
             acc_sh, zb_v, rows_v, xcr_v, eid_v, srci_v, idx_v, cnts_v, sm, sem) = refs
        else:
            (wf_hbm, srcs_hbm, segs_hbm, eids_hbm, cnts_hbm, y_hbm,
             acc_sh, zb_v, rows_v, eid_v, srci_v, idx_v, cnts_v, sm, sem) = refs
        c = lax.axis_index("c")
        s = lax.axis_index("s")
        lanes = lax.iota(jnp.int32, 16)
        pltpu.sync_copy(cnts_hbm, cnts_v.at[pl.ds(0, NW * 32)])
        gb = jnp.int32(0)
        for b in range(NB):
            sm[b] = gb
            gb = gb + lax.fori_loop(0, NW, lambda wp, a: a + _cv(cnts_v, wp * 32 + b), jnp.int32(0))
        sm[NB] = gb

        zf = jnp.zeros((16,), jnp.float32)

        def zrow(i, _):
            for cc in range(8):
                zb_v[i, pl.ds(cc * 16, 16)] = zf
            return _

        lax.fori_loop(0, 128, zrow, None)

        for j in range(NB // 2):
            b = 2 * j + c
            A = sm[b]
            Bnd = sm[b + 1]
            ln = Bnd - A
            for k in range(4):
                pltpu.sync_copy(zb_v, acc_sh.at[pl.ds(s * 512 + k * 128, 128)])
            plsc.subcore_barrier()
            nch = lax.shift_right_logical(ln + 127, 7)
            my = lax.shift_right_logical(nch - s + 15, 4)

            def chunk(jj, _, A=A, Bnd=Bnd, b=b):
                st = pl.multiple_of(A + (s + jj * 16) * 128, 8)
                pltpu.sync_copy(eids_hbm.at[pl.ds(st, 128)], eid_v)
                pltpu.sync_copy(segs_hbm.at[pl.ds(st, 128)], idx_v)
                if with_xc:
                    pltpu.sync_copy(srcs_hbm.at[pl.ds(st, 128)], srci_v)
                for t in range(8):
                    pos = st + t * 16 + lanes
                    ok = pos < Bnd
                    ev = eid_v[pl.ds(t * 16, 16)]
                    eid_v[pl.ds(t * 16, 16)] = jnp.where(ok, ev, 0)
                    sv = idx_v[pl.ds(t * 16, 16)]
                    idx_v[pl.ds(t * 16, 16)] = jnp.where(ok, sv - b * SEGB, SEGB)
                    if with_xc:
                        rv = srci_v[pl.ds(t * 16, 16)]
                        srci_v[pl.ds(t * 16, 16)] = jnp.where(ok, rv, 0)
                pltpu.async_copy(wf_hbm.at[eid_v], rows_v, sem).wait()
                if with_xc:
                    pltpu.async_copy(xc_hbm.at[srci_v], xcr_v, sem).wait()

                    def mul(r, _):
                        for cc in range(8):
                            rows_v[r, pl.ds(cc * 16, 16)] = (
                                rows_v[r, pl.ds(cc * 16, 16)] * xcr_v[r, pl.ds(cc * 16, 16)])
                        return _

                    lax.fori_loop(0, 128, mul, None, unroll=4)
                pltpu.sync_copy(rows_v, acc_sh.at[idx_v], add=True)
                return _

            lax.fori_loop(0, my, chunk, None)
            plsc.subcore_barrier()
            for k in range(4):
                pltpu.sync_copy(acc_sh.at[pl.ds(s * 512 + k * 128, 128)],
                                y_hbm.at[pl.ds(pl.multiple_of(b * SEGB + s * 512 + k * 128, 8), 128)])
            plsc.subcore_barrier()

    return body


def _segsum_call(with_xc):
    scratch = [
        pltpu.VMEM_SHARED((SEGB + 128, H), jnp.float32),
        pltpu.VMEM((128, H), jnp.float32),
        pltpu.VMEM((128, H), jnp.float32),
    ]
    if with_xc:
        scratch.append(pltpu.VMEM((128, H), jnp.float32))
    scratch += [
        pltpu.VMEM((128,), jnp.int32),
        pltpu.VMEM((128,), jnp.int32),
        pltpu.VMEM((128,), jnp.int32),
        pltpu.VMEM((NW * 32 + 16,), jnp.int32),
        pltpu.SMEM((64,), jnp.int32),
        pltpu.SemaphoreType.DMA,
    ]
    return functools.partial(
        pl.kernel,
        _make_segsum(with_xc),
        out_type=jax.ShapeDtypeStruct((NB * SEGB, H), jnp.float32),
        mesh=_MESH,
        scratch_types=scratch,
        compiler_params=pltpu.CompilerParams(needs_layout_passes=False),
    )()


@jax.jit
def _sc_segsum_plain(wf, srcs, segs, eids, cnts):
    return _segsum_call(False)(wf, srcs, segs, eids, cnts)


@jax.jit
def _sc_segsum_mul(wf, xc, srcs, segs, eids, cnts):
    return _segsum_call(True)(wf, xc, srcs, segs, eids, cnts)


def _edge_dense_body(d2_ref, means_ref, betas_ref, dproj_ref, dprojb_ref,
                     f1_ref, f1b_ref, f2_ref, f2b_ref,
                     wn_ref, wf0_ref, wf1_ref, wf2_ref):
    d2 = d2_ref[...].reshape(1024)
    d = jnp.sqrt(d2 + 1e-12)
    rbf = jnp.exp(-betas_ref[0][None, :] * (jnp.exp(-d)[:, None] - means_ref[0][None, :]) ** 2)
    Cc = (0.5 * (jnp.cos(d * (jnp.pi / CUT_HI)) + 1.0) * (d < CUT_HI))[:, None]
    wn_ref[...] = (jnp.dot(rbf, dproj_ref[...], preferred_element_type=jnp.float32)
                   + dprojb_ref[0][None, :]) * Cc
    for l, out_ref in enumerate((wf0_ref, wf1_ref, wf2_ref)):
        h1 = _silu(jnp.dot(rbf, f1_ref[l], preferred_element_type=jnp.float32) + f1b_ref[l][None, :])
        out_ref[...] = (jnp.dot(h1, f2_ref[l], preferred_element_type=jnp.float32)
                        + f2b_ref[l][None, :]) * Cc


@jax.jit
def _tc_edge_dense(d2p2, means, betas, dproj, dprojb, f1, f1b, f2, f2b):
    eo = jax.ShapeDtypeStruct((E, H), jnp.float32)
    full = lambda *shape: pl.BlockSpec(shape, lambda i: (0,) * len(shape))
    return pl.pallas_call(
        _edge_dense_body,
        grid=(157,),
        in_specs=[
            pl.BlockSpec((8, 128), lambda i: (i, 0)),
            full(1, NRBF), full(1, NRBF), full(NRBF, H), full(1, H),
            full(L, NRBF, H), full(L, H), full(L, H, H), full(L, H),
        ],
        out_specs=[pl.BlockSpec((1024, H), lambda i: (i, 0))] * 4,
        out_shape=(eo, eo, eo, eo),
    )(d2p2, means, betas, dproj, dprojb, f1, f1b, f2, f2b)


def _ne_body(z_ref, cntp_ref, wt_ref, emb_ref, ne_emb_ref, comb_ref, combb_ref,
             conv0_ref, x_ref, mask_ref, xc_ref):
    zb = z_ref[...]
    oh = (zb == lax.broadcasted_iota(jnp.int32, (1, MAXZ), 1)).astype(jnp.float32)
    xe = jnp.dot(oh, emb_ref[...], preferred_element_type=jnp.float32)
    wtb = wt_ref[...].reshape(128, MAXZ, H)
    agg = jnp.sum(wtb * ne_emb_ref[...][None, :, :], axis=1)
    x = (jnp.dot(xe, comb_ref[0:H, :], preferred_element_type=jnp.float32)
         + jnp.dot(agg, comb_ref[H:, :], preferred_element_type=jnp.float32)
         + combb_ref[0][None, :])
    x_ref[...] = x
    cnt = jnp.sum(cntp_ref[...], axis=0)[0]
    mask_ref[...] = (cnt > 0).astype(jnp.float32)
    xc_ref[...] = jnp.dot(x, conv0_ref[...], preferred_element_type=jnp.float32)


@jax.jit
def _tc_ne(z2, cntp4, wt, emb, ne_emb, comb, combb, conv0):
    full = lambda *shape: pl.BlockSpec(shape, lambda i: (0,) * len(shape))
    return pl.pallas_call(
        _ne_body,
        grid=(79,),
        in_specs=[
            pl.BlockSpec((128, 1), lambda i: (i, 0)),
            pl.BlockSpec((NW, 1, 128, MAXZ), lambda i: (0, i // 4, i % 4, 0)),
            pl.BlockSpec((2048, H), lambda i: (i, 0)),
            full(MAXZ, H), full(MAXZ, H), full(2 * H, H), full(1, H), full(H, H),
        ],
        out_specs=[
            pl.BlockSpec((128, H), lambda i: (i, 0)),
            pl.BlockSpec((128, MAXZ), lambda i: (i, 0)),
            pl.BlockSpec((128, H), lambda i: (i, 0)),
        ],
        out_shape=(
            jax.ShapeDtypeStruct((10112, H), jnp.float32),
            jax.ShapeDtypeStruct((10112, MAXZ), jnp.float32),
            jax.ShapeDtypeStruct((10112, H), jnp.float32),
        ),
    )(z2, cntp4, wt, emb, ne_emb, comb, combb, conv0)


def _make_attn_body(has_next):
    def body(*refs):
        if has_next:
            (y_ref, x_ref, mask_ref, qw, qb, kw, kb, vw, vb, ow, ob,
             blkw, blkb, convw, xo_ref, xco_ref) = refs
        else:
            (y_ref, x_ref, mask_ref, qw, qb, kw, kb, vw, vb, ow, ob,
             blkw, blkb, xo_ref) = refs
        yb = y_ref[...]
        q = (jnp.dot(yb, qw[...], preferred_element_type=jnp.float32) + qb[0][None, :]).reshape(128, MAXZ, NH, DH)
        k = (jnp.dot(yb, kw[...], preferred_element_type=jnp.float32) + kb[0][None, :]).reshape(128, MAXZ, NH, DH)
        v = (jnp.dot(yb, vw[...], preferred_element_type=jnp.float32) + vb[0][None, :]).reshape(128, MAXZ, NH, DH)
        maskb = mask_ref[...]
        outs = []
        for h in range(NH):
            qh = q[:, :, h, :]
            kh = k[:, :, h, :]
            vh = v[:, :, h, :]
            S = jnp.sum(qh[:, :, None, :] * kh[:, None, :, :], axis=-1)
            S = jnp.where(maskb[:, None, :] > 0, S, jnp.float32(-1e9))
            S = S - jnp.max(S, axis=-1, keepdims=True)
            p = jnp.exp(S)
            p = p / jnp.sum(p, axis=-1, keepdims=True)
            outs.append(jnp.sum(p[:, :, :, None] * vh[:, None, :, :], axis=2))
        out = jnp.stack(outs, axis=2).reshape(128 * MAXZ, H)
        out = jnp.dot(out, ow[...], preferred_element_type=jnp.float32) + ob[0][None, :]
        out = out.reshape(128, MAXZ, H) * maskb[:, :, None]
        yn = _silu(jnp.sum(out, axis=1))
        xn = x_ref[...] + jnp.dot(yn, blkw[...], preferred_element_type=jnp.float32) + blkb[0][None, :]
        xo_ref[...] = xn
        if has_next:
            xco_ref[...] = jnp.dot(xn, convw[...], preferred_element_type=jnp.float32)
    return body


def _tc_attn(y, x, maskf, qw, qb, kw, kb, vw, vb, ow, ob, blkw, blkb, convw):
    has_next = convw is not None
    full = lambda *shape: pl.BlockSpec(shape, lambda i: (0,) * len(shape))
    w128 = full(H, H)
    b128 = full(1, H)
    in_specs = [
        pl.BlockSpec((2048, H), lambda i: (i, 0)),
        pl.BlockSpec((128, H), lambda i: (i, 0)),
        pl.BlockSpec((128, MAXZ), lambda i: (i, 0)),
        w128, b128, w128, b128, w128, b128, w128, b128, w128, b128,
    ]
    outs = [pl.BlockSpec((128, H), lambda i: (i, 0))]
    out_shape = [jax.ShapeDtypeStruct((10112, H), jnp.float32)]
    args = [y, x, maskf, qw, qb, kw, kb, vw, vb, ow, ob, blkw, blkb]
    if has_next:
        in_specs.append(w128)
        outs.append(pl.BlockSpec((128, H), lambda i: (i, 0)))
        out_shape.append(jax.ShapeDtypeStruct((10112, H), jnp.float32))
        args.append(convw)
    return pl.pallas_call(
        _make_attn_body(has_next),
        grid=(79,),
        in_specs=in_specs,
        out_specs=outs,
        out_shape=tuple(out_shape),
        compiler_params=pltpu.CompilerParams(vmem_limit_bytes=100 * 1024 * 1024),
    )(*args)


def kernel(z, pos, edge_index, emb, ne_emb, ne_dproj_w, ne_dproj_b, ne_comb_w, ne_comb_b, rbf_means, rbf_betas, conv_lin1_w, filt1_w, filt1_b, filt2_w, filt2_b, q_w, q_b, k_w, k_b, v_w, v_b, o_w, o_b, blk_w, blk_b):
    ei_p = jnp.pad(edge_index.astype(jnp.int32), ((0, 0), (0, EPAD - E)))
    posf = pos.reshape(-1)
    zi = z.astype(jnp.int32)
    d2p, segp, cnts = _sc_prep(ei_p[0], ei_p[1], posf, zi)
    cnts1d = cnts.reshape(-1)
    srcs, segs, eids, cntp = _sc_place(ei_p[0], ei_p[1], segp, cnts1d)
    wn, wf0, wf1, wf2 = _tc_edge_dense(
        d2p.reshape(1252, 128), rbf_means.reshape(1, NRBF), rbf_betas.reshape(1, NRBF),
        ne_dproj_w, ne_dproj_b.reshape(1, H), filt1_w, filt1_b, filt2_w, filt2_b)
    wt = _sc_segsum_plain(wn, srcs, segs, eids, cnts1d)
    xp, maskp, xc = _tc_ne(
        zi.reshape(N, 1), cntp.reshape(NW, NB, 513, MAXZ), wt, emb, ne_emb,
        ne_comb_w, ne_comb_b.reshape(1, H), conv_lin1_w[0])
    wfs = (wf0, wf1, wf2)
    for l in range(L):
        y = _sc_segsum_mul(wfs[l], xc, srcs, segs, eids, cnts1d)
        convw = conv_lin1_w[l + 1] if l < L - 1 else None
        rs = _tc_attn(y, xp, maskp,
                      q_w[l], q_b[l].reshape(1, H), k_w[l], k_b[l].reshape(1, H),
                      v_w[l], v_b[l].reshape(1, H), o_w[l], o_b[l].reshape(1, H),
                      blk_w[l], blk_b[l].reshape(1, H), convw)
        if l < L - 1:
            xp, xc = rs
        else:
            xp = rs[0]
    return xp[:N]


# speedup vs baseline: 1.5981x; 1.5981x over previous
import functools

import jax
import jax.numpy as jnp
from jax import lax
from jax.experimental import pallas as pl
from jax.experimental.pallas import tpu as pltpu
from jax.experimental.pallas import tpu_sc as plsc

N = 10000
E = 160000
H = 128
NH = 8
DH = H // NH
L = 3
NRBF = 50
MAXZ = 16
CUT_HI = 5.0

NB = 20
SEGB = 8192
NW = 32
CH = 5008
EPAD = 160256
EALLOC = 165120

_MESH = plsc.VectorSubcoreMesh(
    core_axis_name="c", subcore_axis_name="s", num_cores=2, num_subcores=16)


def _cv(ref, i):
    return ref[pl.ds(i, 16)][0]


def _silu(x):
    return x * jax.nn.sigmoid(x)


def _cosine_cutoff(d):
    return 0.5 * (jnp.cos(d * jnp.pi / CUT_HI) + 1.0) * (d < CUT_HI)


def _expnorm_rbf(d, means, betas):
    return jnp.exp(-betas[None, :] * (jnp.exp(-d[:, None]) - means[None, :]) ** 2)


def _prep_body(srce_hbm, dste_hbm, posf_hbm, z_hbm, d2_hbm, seg_hbm, cnt_hbm,
               posf_v, z_v, src_v, dst_v, d2_v, segv_v, cw_v, sm):
    c = lax.axis_index("c")
    s = lax.axis_index("s")
    w = s * 2 + c
    start = w * CH
    nume = jnp.where(w == NW - 1, E - (NW - 1) * CH, CH)
    nv = nume // 16

    pltpu.sync_copy(posf_hbm, posf_v)
    pltpu.sync_copy(z_hbm, z_v)
    pltpu.sync_copy(srce_hbm.at[pl.ds(start, CH)], src_v)
    pltpu.sync_copy(dste_hbm.at[pl.ds(start, CH)], dst_v)

    lanes = lax.iota(jnp.int32, 16)

    def vec_body(i, carry):
        cv0, cv1 = carry
        srcv = src_v[pl.ds(i * 16, 16)]
        dstv = dst_v[pl.ds(i * 16, 16)]
        s3 = srcv * 3
        t3 = dstv * 3
        dx = plsc.load_gather(posf_v, [s3]) - plsc.load_gather(posf_v, [t3])
        dy = plsc.load_gather(posf_v, [s3 + 1]) - plsc.load_gather(posf_v, [t3 + 1])
        dz = plsc.load_gather(posf_v, [s3 + 2]) - plsc.load_gather(posf_v, [t3 + 2])
        d2 = dx * dx + dy * dy + dz * dz
        zsv = plsc.load_gather(z_v, [srcv])
        d2_v[pl.ds(i * 16, 16)] = d2
        segv_v[pl.ds(i * 16, 16)] = dstv * MAXZ + zsv
        bv = lax.shift_right_logical(dstv, 9)
        for b in range(16):
            cv0 = cv0 + jnp.sum((bv == b).astype(jnp.int32)) * (lanes == b).astype(jnp.int32)
        for b in range(16, NB):
            cv1 = cv1 + jnp.sum((bv == b).astype(jnp.int32)) * (lanes == (b - 16)).astype(jnp.int32)
        return cv0, cv1

    zv16 = jnp.zeros((16,), jnp.int32)
    cv0, cv1 = lax.fori_loop(0, nv, vec_body, (zv16, zv16))
    pad8 = lambda v: lax.shift_left(lax.shift_right_logical(v + 7, 3), 3)
    cw_v[pl.ds(0, 16)] = pad8(cv0)
    cw_v[pl.ds(16, 16)] = jnp.where(lanes < NB - 16, pad8(cv1), 0)
    pltpu.sync_copy(cw_v, cnt_hbm.at[w])
    pltpu.sync_copy(d2_v, d2_hbm.at[pl.ds(start, CH)])
    pltpu.sync_copy(segv_v, seg_hbm.at[pl.ds(start, CH)])


@jax.jit
def _sc_prep(srcp, dstp, posf, z):
    return pl.kernel(
        _prep_body,
        out_type=(
            jax.ShapeDtypeStruct((EPAD,), jnp.float32),
            jax.ShapeDtypeStruct((EPAD,), jnp.int32),
            jax.ShapeDtypeStruct((NW, 32), jnp.int32),
        ),
        mesh=_MESH,
        scratch_types=[
            pltpu.VMEM((N * 3,), jnp.float32),
            pltpu.VMEM((N,), jnp.int32),
            pltpu.VMEM((CH,), jnp.int32),
            pltpu.VMEM((CH,), jnp.int32),
            pltpu.VMEM((CH,), jnp.float32),
            pltpu.VMEM((CH,), jnp.int32),
            pltpu.VMEM((32,), jnp.int32),
            pltpu.SMEM((32,), jnp.int32),
        ],
        compiler_params=pltpu.CompilerParams(needs_layout_passes=False),
    )(srcp, dstp, posf, z)


def _place_body(srce_hbm, dste_hbm, sege_hbm, cnts_hbm,
                srcs_hbm, segs_hbm, eids_hbm, cntp_hbm,
                src_v, dst_v, seg_v, bsrc_v, bseg_v, beid_v, cnts_v, cnt_loc,
                sm):
    c = lax.axis_index("c")
    s = lax.axis_index("s")
    w = s * 2 + c
    start = w * CH
    nume = jnp.where(w == NW - 1, E - (NW - 1) * CH, CH)
    nv = nume // 16
    lanes = lax.iota(jnp.int32, 16)

    pltpu.sync_copy(srce_hbm.at[pl.ds(start, CH)], src_v)
    pltpu.sync_copy(dste_hbm.at[pl.ds(start, CH)], dst_v)
    pltpu.sync_copy(sege_hbm.at[pl.ds(start, CH)], seg_v)
    pltpu.sync_copy(cnts_hbm, cnts_v.at[pl.ds(0, NW * 32)])

    loc = jnp.int32(0)
    gb = jnp.int32(0)
    for b in range(NB):
        sm[b] = loc
        sm[40 + b] = loc
        loc = loc + _cv(cnts_v, w * 32 + b)
        part = lax.fori_loop(0, w, lambda wp, a: a + _cv(cnts_v, wp * 32 + b), jnp.int32(0))
        tot = lax.fori_loop(0, NW, lambda wp, a: a + _cv(cnts_v, wp * 32 + b), jnp.int32(0))
        sm[20 + b] = gb + part
        gb = gb + tot

    def vec_body(i, _):
        srcv = src_v[pl.ds(i * 16, 16)]
        dstv = dst_v[pl.ds(i * 16, 16)]
        segv = seg_v[pl.ds(i * 16, 16)]
        eidv = start + i * 16 + lanes
        bv = lax.shift_right_logical(dstv, 9)
        for b in range(NB):
            m = bv == b
            off = sm[40 + b]
            plsc.store_compressed(bsrc_v.at[pl.ds(off, 16)], srcv, mask=m)
            plsc.store_compressed(bseg_v.at[pl.ds(off, 16)], segv, mask=m)
            plsc.store_compressed(beid_v.at[pl.ds(off, 16)], eidv, mask=m)
            sm[40 + b] = off + jnp.sum(m.astype(jnp.int32))
        return _

    lax.fori_loop(0, nv, vec_body, None)

    z16 = jnp.zeros((16,), jnp.int32)
    for b in range(NB):
        cur = sm[40 + b]
        end = sm[b] + _cv(cnts_v, w * 32 + b)
        m = lanes < (end - cur)
        plsc.store_compressed(bsrc_v.at[pl.ds(cur, 16)], z16, mask=m)
        plsc.store_compressed(bseg_v.at[pl.ds(cur, 16)],
                              jnp.full((16,), b * SEGB + SEGB, jnp.int32), mask=m)
        plsc.store_compressed(beid_v.at[pl.ds(cur, 16)], z16, mask=m)

    for b in range(NB):
        lo = sm[b]
        gbase = sm[20 + b]
        pcnt = _cv(cnts_v, w * 32 + b)
        n128 = lax.shift_right_logical(pcnt, 7)
        n8 = lax.shift_right_logical(pcnt - n128 * 128, 3)

        def big(j, _, lo=lo, gbase=gbase):
            go = pl.multiple_of(gbase + j * 128, 8)
            lo8 = pl.multiple_of(lo + j * 128, 8)
            pltpu.sync_copy(bsrc_v.at[pl.ds(lo8, 128)], srcs_hbm.at[pl.ds(go, 128)])
            pltpu.sync_copy(bseg_v.at[pl.ds(lo8, 128)], segs_hbm.at[pl.ds(go, 128)])
            pltpu.sync_copy(beid_v.at[pl.ds(lo8, 128)], eids_hbm.at[pl.ds(go, 128)])
            return _

        def small(j, _, lo=lo, gbase=gbase, n128=n128):
            o = n128 * 128 + j * 8
            go = pl.multiple_of(gbase + o, 8)
            lo8 = pl.multiple_of(lo + o, 8)
            pltpu.sync_copy(bsrc_v.at[pl.ds(lo8, 8)], srcs_hbm.at[pl.ds(go, 8)])
            pltpu.sync_copy(bseg_v.at[pl.ds(lo8, 8)], segs_hbm.at[pl.ds(go, 8)])
            pltpu.sync_copy(beid_v.at[pl.ds(lo8, 8)], eids_hbm.at[pl.ds(go, 8)])
            return _

        lax.fori_loop(0, n128, big, None)
        lax.fori_loop(0, n8, small, None)

    ones16 = jnp.ones((16,), jnp.int32)
    for b in range(NB):
        lo = sm[b]
        pcnt = _cv(cnts_v, w * 32 + b)

        def zero(j, _):
            cnt_loc[pl.ds(j * 16, 16)] = z16
            return _

        lax.fori_loop(0, 513, zero, None)

        def count(j, _, lo=lo, pcnt=pcnt):
            sv = bseg_v[pl.ds(lo + j * 16, 16)]
            idx = sv - b * SEGB
            m = lanes < (pcnt - j * 16)
            plsc.addupdate_scatter(cnt_loc, [idx], ones16, mask=m)
            return _

        lax.fori_loop(0, lax.shift_right_logical(pcnt + 15, 4), count, None)
        pltpu.sync_copy(cnt_loc, cntp_hbm.at[pl.ds(pl.multiple_of(w * (NB * 8208) + b * 8208, 8), 8208)])


@jax.jit
def _sc_place(srcp, dstp, segp, cnts):
    return pl.kernel(
        _place_body,
        out_type=(
            jax.ShapeDtypeStruct((EALLOC,), jnp.int32),
            jax.ShapeDtypeStruct((EALLOC,), jnp.int32),
            jax.ShapeDtypeStruct((EALLOC,), jnp.int32),
            jax.ShapeDtypeStruct((NW * NB * 8208,), jnp.int32),
        ),
        mesh=_MESH,
        scratch_types=[
            pltpu.VMEM((CH,), jnp.int32),
            pltpu.VMEM((CH,), jnp.int32),
            pltpu.VMEM((CH,), jnp.int32),
            pltpu.VMEM((5184,), jnp.int32),
            pltpu.VMEM((5184,), jnp.int32),
            pltpu.VMEM((5184,), jnp.int32),
            pltpu.VMEM((NW * 32 + 16,), jnp.int32),
            pltpu.VMEM((8208,), jnp.int32),
            pltpu.SMEM((64,), jnp.int32),
        ],
        compiler_params=pltpu.CompilerParams(needs_layout_passes=False),
    )(srcp, dstp, segp, cnts)


def _make_segsum(with_xc):
    def body(*refs):
        if with_xc:
            (wf_hbm, xc_hbm, srcs_hbm, segs_hbm, eids_hbm, cnts_hbm, y_hbm,
             acc_sh, zb_v, rows_v, xcr_v, eid_v, srci_v, idx_v, cnts_v, sm, sem) = refs
        else:
            (wf_hbm, srcs_hbm, segs_hbm, eids_hbm, cnts_hbm, y_hbm,
             acc_sh, zb_v, rows_v, eid_v, srci_v, idx_v, cnts_v, sm, sem) = refs
        c = lax.axis_index("c")
        s = lax.axis_index("s")
        lanes = lax.iota(jnp.int32, 16)
        pltpu.sync_copy(cnts_hbm, cnts_v.at[pl.ds(0, NW * 32)])
        gb = jnp.int32(0)
        for b in range(NB):
            sm[b] = gb
            gb = gb + lax.fori_loop(0, NW, lambda wp, a: a + _cv(cnts_v, wp * 32 + b), jnp.int32(0))
        sm[NB] = gb

        zf = jnp.zeros((16,), jnp.float32)

        def zrow(i, _):
            for cc in range(8):
                zb_v[i, pl.ds(cc * 16, 16)] = zf
            return _

        lax.fori_loop(0, 128, zrow, None)

        for j in range(NB // 2):
            b = 2 * j + c
            A = sm[b]
            Bnd = sm[b + 1]
            ln = Bnd - A
            for k in range(4):
                pltpu.sync_copy(zb_v, acc_sh.at[pl.ds(s * 512 + k * 128, 128)])
            plsc.subcore_barrier()
            nch = lax.shift_right_logical(ln + 127, 7)
            my = lax.shift_right_logical(nch - s + 15, 4)

            def chunk(jj, _, A=A, Bnd=Bnd, b=b):
                st = pl.multiple_of(A + (s + jj * 16) * 128, 8)
                pltpu.sync_copy(eids_hbm.at[pl.ds(st, 128)], eid_v)
                pltpu.sync_copy(segs_hbm.at[pl.ds(st, 128)], idx_v)
                if with_xc:
                    pltpu.sync_copy(srcs_hbm.at[pl.ds(st, 128)], srci_v)
                for t in range(8):
                    pos = st + t * 16 + lanes
                    ok = pos < Bnd
                    ev = eid_v[pl.ds(t * 16, 16)]
                    eid_v[pl.ds(t * 16, 16)] = jnp.where(ok, ev, 0)
                    sv = idx_v[pl.ds(t * 16, 16)]
                    idx_v[pl.ds(t * 16, 16)] = jnp.where(ok, sv - b * SEGB, SEGB)
                    if with_xc:
                        rv = srci_v[pl.ds(t * 16, 16)]
                        srci_v[pl.ds(t * 16, 16)] = jnp.where(ok, rv, 0)
                pltpu.async_copy(wf_hbm.at[eid_v], rows_v, sem).wait()
                if with_xc:
                    pltpu.async_copy(xc_hbm.at[srci_v], xcr_v, sem).wait()

                    def mul(r, _):
                        for cc in range(8):
                            rows_v[r, pl.ds(cc * 16, 16)] = (
                                rows_v[r, pl.ds(cc * 16, 16)] * xcr_v[r, pl.ds(cc * 16, 16)])
                        return _

                    lax.fori_loop(0, 128, mul, None, unroll=4)
                pltpu.sync_copy(rows_v, acc_sh.at[idx_v], add=True)
                return _

            lax.fori_loop(0, my, chunk, None)
            plsc.subcore_barrier()
            for k in range(4):
                pltpu.sync_copy(acc_sh.at[pl.ds(s * 512 + k * 128, 128)],
                                y_hbm.at[pl.ds(pl.multiple_of(b * SEGB + s * 512 + k * 128, 8), 128)])
            plsc.subcore_barrier()

    return body


def _segsum_call(with_xc):
    scratch = [
        pltpu.VMEM_SHARED((SEGB + 128, H), jnp.float32),
        pltpu.VMEM((128, H), jnp.float32),
        pltpu.VMEM((128, H), jnp.float32),
    ]
    if with_xc:
        scratch.append(pltpu.VMEM((128, H), jnp.float32))
    scratch += [
        pltpu.VMEM((128,), jnp.int32),
        pltpu.VMEM((128,), jnp.int32),
        pltpu.VMEM((128,), jnp.int32),
        pltpu.VMEM((NW * 32 + 16,), jnp.int32),
        pltpu.SMEM((64,), jnp.int32),
        pltpu.SemaphoreType.DMA,
    ]
    return functools.partial(
        pl.kernel,
        _make_segsum(with_xc),
        out_type=jax.ShapeDtypeStruct((NB * SEGB, H), jnp.float32),
        mesh=_MESH,
        scratch_types=scratch,
        compiler_params=pltpu.CompilerParams(needs_layout_passes=False),
    )()


@jax.jit
def _sc_segsum_plain(wf, srcs, segs, eids, cnts):
    return _segsum_call(False)(wf, srcs, segs, eids, cnts)


@jax.jit
def _sc_segsum_mul(wf, xc, srcs, segs, eids, cnts):
    return _segsum_call(True)(wf, xc, srcs, segs, eids, cnts)


def _edge_dense_body(d2_ref, means_ref, betas_ref, dproj_ref, dprojb_ref,
                     f1_ref, f1b_ref, f2_ref, f2b_ref,
                     wn_ref, wf0_ref, wf1_ref, wf2_ref):
    d2 = d2_ref[...].reshape(1024)
    d = jnp.sqrt(d2 + 1e-12)
    rbf = jnp.exp(-betas_ref[0][None, :] * (jnp.exp(-d)[:, None] - means_ref[0][None, :]) ** 2)
    Cc = (0.5 * (jnp.cos(d * (jnp.pi / CUT_HI)) + 1.0) * (d < CUT_HI))[:, None]
    wn_ref[...] = (jnp.dot(rbf, dproj_ref[...], preferred_element_type=jnp.float32)
                   + dprojb_ref[0][None, :]) * Cc
    for l, out_ref in enumerate((wf0_ref, wf1_ref, wf2_ref)):
        h1 = _silu(jnp.dot(rbf, f1_ref[l], preferred_element_type=jnp.float32) + f1b_ref[l][None, :])
        out_ref[...] = (jnp.dot(h1, f2_ref[l], preferred_element_type=jnp.float32)
                        + f2b_ref[l][None, :]) * Cc


@jax.jit
def _tc_edge_dense(d2p2, means, betas, dproj, dprojb, f1, f1b, f2, f2b):
    eo = jax.ShapeDtypeStruct((E, H), jnp.float32)
    full = lambda *shape: pl.BlockSpec(shape, lambda i: (0,) * len(shape))
    return pl.pallas_call(
        _edge_dense_body,
        grid=(157,),
        in_specs=[
            pl.BlockSpec((8, 128), lambda i: (i, 0)),
            full(1, NRBF), full(1, NRBF), full(NRBF, H), full(1, H),
            full(L, NRBF, H), full(L, H), full(L, H, H), full(L, H),
        ],
        out_specs=[pl.BlockSpec((1024, H), lambda i: (i, 0))] * 4,
        out_shape=(eo, eo, eo, eo),
    )(d2p2, means, betas, dproj, dprojb, f1, f1b, f2, f2b)


def _ne_body(z_ref, cntp_ref, wt_ref, emb_ref, ne_emb_ref, comb_ref, combb_ref,
             conv0_ref, x_ref, mask_ref, xc_ref):
    zb = z_ref[...]
    oh = (zb == lax.broadcasted_iota(jnp.int32, (1, MAXZ), 1)).astype(jnp.float32)
    xe = jnp.dot(oh, emb_ref[...], preferred_element_type=jnp.float32)
    wtb = wt_ref[...].reshape(128, MAXZ, H)
    agg = jnp.sum(wtb * ne_emb_ref[...][None, :, :], axis=1)
    x = (jnp.dot(xe, comb_ref[0:H, :], preferred_element_type=jnp.float32)
         + jnp.dot(agg, comb_ref[H:, :], preferred_element_type=jnp.float32)
         + combb_ref[0][None, :])
    x_ref[...] = x
    cnt = jnp.sum(cntp_ref[...], axis=0)[0]
    mask_ref[...] = (cnt > 0).astype(jnp.float32)
    xc_ref[...] = jnp.dot(x, conv0_ref[...], preferred_element_type=jnp.float32)


@jax.jit
def _tc_ne(z2, cntp4, wt, emb, ne_emb, comb, combb, conv0):
    full = lambda *shape: pl.BlockSpec(shape, lambda i: (0,) * len(shape))
    return pl.pallas_call(
        _ne_body,
        grid=(79,),
        in_specs=[
            pl.BlockSpec((128, 1), lambda i: (i, 0)),
            pl.BlockSpec((NW, 1, 128, MAXZ), lambda i: (0, i // 4, i % 4, 0)),
            pl.BlockSpec((2048, H), lambda i: (i, 0)),
            full(MAXZ, H), full(MAXZ, H), full(2 * H, H), full(1, H), full(H, H),
        ],
        out_specs=[
            pl.BlockSpec((128, H), lambda i: (i, 0)),
            pl.BlockSpec((128, MAXZ), lambda i: (i, 0)),
            pl.BlockSpec((128, H), lambda i: (i, 0)),
        ],
        out_shape=(
            jax.ShapeDtypeStruct((10112, H), jnp.float32),
            jax.ShapeDtypeStruct((10112, MAXZ), jnp.float32),
            jax.ShapeDtypeStruct((10112, H), jnp.float32),
        ),
    )(z2, cntp4, wt, emb, ne_emb, comb, combb, conv0)


def _make_attn_body(has_next):
    def body(*refs):
        if has_next:
            (y_ref, x_ref, mask_ref, qw, qb, kw, kb, vw, vb, ow, ob,
             blkw, blkb, convw, xo_ref, xco_ref) = refs
        else:
            (y_ref, x_ref, mask_ref, qw, qb, kw, kb, vw, vb, ow, ob,
             blkw, blkb, xo_ref) = refs
        yb = y_ref[...]
        q = jnp.dot(yb, qw[...], preferred_element_type=jnp.float32) + qb[0][None, :]
        k = jnp.dot(yb, kw[...], preferred_element_type=jnp.float32) + kb[0][None, :]
        v = jnp.dot(yb, vw[...], preferred_element_type=jnp.float32) + vb[0][None, :]
        maskb = mask_ref[...]
        q3 = q.reshape(128, MAXZ, H)
        k3 = k.reshape(128, MAXZ, H)
        v3 = v.reshape(128, MAXZ, H)
        colh = lax.broadcasted_iota(jnp.int32, (H, NH), 0) // DH
        rowh = lax.broadcasted_iota(jnp.int32, (H, NH), 1)
        bd = (colh == rowh).astype(jnp.float32)
        hid = lax.broadcasted_iota(jnp.int32, (NH, H), 0)
        cid = lax.broadcasted_iota(jnp.int32, (NH, H), 1)
        bdt = (cid // MAXZ == hid).astype(jnp.float32)
        S = jnp.zeros((128 * MAXZ, H), jnp.float32)
        for kk in range(MAXZ):
            t = (q3 * k3[:, kk, :][:, None, :]).reshape(128 * MAXZ, H)
            sk = jnp.dot(t, bd, preferred_element_type=jnp.float32)
            sel = (cid == hid * MAXZ + kk).astype(jnp.float32)
            S = S + jnp.dot(sk, sel, preferred_element_type=jnp.float32)
        maskcol = jnp.concatenate([maskb] * NH, axis=1)
        m2048 = jnp.broadcast_to(maskcol[:, None, :], (128, MAXZ, H)).reshape(128 * MAXZ, H)
        S = jnp.where(m2048 > 0, S, jnp.float32(-1e9))
        P3 = S.reshape(128 * MAXZ, NH, MAXZ)
        P3 = P3 - jnp.max(P3, axis=-1, keepdims=True)
        P3 = jnp.exp(P3)
        P3 = P3 / jnp.sum(P3, axis=-1, keepdims=True)
        out = jnp.zeros((128 * MAXZ, H), jnp.float32)
        for kk in range(MAXZ):
            pk = P3[:, :, kk]
            pk128 = jnp.dot(pk, bdt, preferred_element_type=jnp.float32)
            vkb = jnp.broadcast_to(v3[:, kk, :][:, None, :], (128, MAXZ, H)).reshape(128 * MAXZ, H)
            out = out + pk128 * vkb
        out = jnp.dot(out, ow[...], preferred_element_type=jnp.float32) + ob[0][None, :]
        out = out.reshape(128, MAXZ, H) * maskb[:, :, None]
        yn = _silu(jnp.sum(out, axis=1))
        xn = x_ref[...] + jnp.dot(yn, blkw[...], preferred_element_type=jnp.float32) + blkb[0][None, :]
        xo_ref[...] = xn
        if has_next:
            xco_ref[...] = jnp.dot(xn, convw[...], preferred_element_type=jnp.float32)
    return body


def _tc_attn(y, x, maskf, qw, qb, kw, kb, vw, vb, ow, ob, blkw, blkb, convw):
    has_next = convw is not None
    full = lambda *shape: pl.BlockSpec(shape, lambda i: (0,) * len(shape))
    w128 = full(H, H)
    b128 = full(1, H)
    in_specs = [
        pl.BlockSpec((2048, H), lambda i: (i, 0)),
        pl.BlockSpec((128, H), lambda i: (i, 0)),
        pl.BlockSpec((128, MAXZ), lambda i: (i, 0)),
        w128, b128, w128, b128, w128, b128, w128, b128, w128, b128,
    ]
    outs = [pl.BlockSpec((128, H), lambda i: (i, 0))]
    out_shape = [jax.ShapeDtypeStruct((10112, H), jnp.float32)]
    args = [y, x, maskf, qw, qb, kw, kb, vw, vb, ow, ob, blkw, blkb]
    if has_next:
        in_specs.append(w128)
        outs.append(pl.BlockSpec((128, H), lambda i: (i, 0)))
        out_shape.append(jax.ShapeDtypeStruct((10112, H), jnp.float32))
        args.append(convw)
    return pl.pallas_call(
        _make_attn_body(has_next),
        grid=(79,),
        in_specs=in_specs,
        out_specs=outs,
        out_shape=tuple(out_shape),
        compiler_params=pltpu.CompilerParams(vmem_limit_bytes=100 * 1024 * 1024),
    )(*args)


def kernel(z, pos, edge_index, emb, ne_emb, ne_dproj_w, ne_dproj_b, ne_comb_w, ne_comb_b, rbf_means, rbf_betas, conv_lin1_w, filt1_w, filt1_b, filt2_w, filt2_b, q_w, q_b, k_w, k_b, v_w, v_b, o_w, o_b, blk_w, blk_b):
    ei_p = jnp.pad(edge_index.astype(jnp.int32), ((0, 0), (0, EPAD - E)))
    posf = pos.reshape(-1)
    zi = z.astype(jnp.int32)
    d2p, segp, cnts = _sc_prep(ei_p[0], ei_p[1], posf, zi)
    cnts1d = cnts.reshape(-1)
    srcs, segs, eids, cntp = _sc_place(ei_p[0], ei_p[1], segp, cnts1d)
    wn, wf0, wf1, wf2 = _tc_edge_dense(
        d2p.reshape(1252, 128), rbf_means.reshape(1, NRBF), rbf_betas.reshape(1, NRBF),
        ne_dproj_w, ne_dproj_b.reshape(1, H), filt1_w, filt1_b, filt2_w, filt2_b)
    wt = _sc_segsum_plain(wn, srcs, segs, eids, cnts1d)
    xp, maskp, xc = _tc_ne(
        zi.reshape(N, 1), cntp.reshape(NW, NB, 513, MAXZ), wt, emb, ne_emb,
        ne_comb_w, ne_comb_b.reshape(1, H), conv_lin1_w[0])
    wfs = (wf0, wf1, wf2)
    for l in range(L):
        y = _sc_segsum_mul(wfs[l], xc, srcs, segs, eids, cnts1d)
        convw = conv_lin1_w[l + 1] if l < L - 1 else None
        rs = _tc_attn(y, xp, maskp,
                      q_w[l], q_b[l].reshape(1, H), k_w[l], k_b[l].reshape(1, H),
                      v_w[l], v_b[l].reshape(1, H), o_w[l], o_b[l].reshape(1, H),
                      blk_w[l], blk_b[l].reshape(1, H), convw)
        if l < L - 1:
            xp, xc = rs
        else:
            xp = rs[0]
    return xp[:N]


# speedup vs baseline: 3.3789x; 2.1143x over previous
import functools

import jax
import jax.numpy as jnp
from jax import lax
from jax.experimental import pallas as pl
from jax.experimental.pallas import tpu as pltpu
from jax.experimental.pallas import tpu_sc as plsc

N = 10000
E = 160000
H = 128
NH = 8
DH = H // NH
L = 3
NRBF = 50
MAXZ = 16
CUT_HI = 5.0

NB = 20
SEGB = 8192
NW = 32
CH = 5008
EPAD = 160256
EALLOC = 165120

_MESH = plsc.VectorSubcoreMesh(
    core_axis_name="c", subcore_axis_name="s", num_cores=2, num_subcores=16)


def _cv(ref, i):
    return ref[pl.ds(i, 16)][0]


def _silu(x):
    return x * jax.nn.sigmoid(x)


def _cosine_cutoff(d):
    return 0.5 * (jnp.cos(d * jnp.pi / CUT_HI) + 1.0) * (d < CUT_HI)


def _expnorm_rbf(d, means, betas):
    return jnp.exp(-betas[None, :] * (jnp.exp(-d[:, None]) - means[None, :]) ** 2)


def _prep_body(srce_hbm, dste_hbm, posf_hbm, z_hbm, d2_hbm, seg_hbm, cnt_hbm,
               posf_v, z_v, src_v, dst_v, d2_v, segv_v, cw_v, sm):
    c = lax.axis_index("c")
    s = lax.axis_index("s")
    w = s * 2 + c
    start = w * CH
    nume = jnp.where(w == NW - 1, E - (NW - 1) * CH, CH)
    nv = nume // 16

    pltpu.sync_copy(posf_hbm, posf_v)
    pltpu.sync_copy(z_hbm, z_v)
    pltpu.sync_copy(srce_hbm.at[pl.ds(start, CH)], src_v)
    pltpu.sync_copy(dste_hbm.at[pl.ds(start, CH)], dst_v)

    lanes = lax.iota(jnp.int32, 16)

    def vec_body(i, carry):
        cv0, cv1 = carry
        srcv = src_v[pl.ds(i * 16, 16)]
        dstv = dst_v[pl.ds(i * 16, 16)]
        s3 = srcv * 3
        t3 = dstv * 3
        dx = plsc.load_gather(posf_v, [s3]) - plsc.load_gather(posf_v, [t3])
        dy = plsc.load_gather(posf_v, [s3 + 1]) - plsc.load_gather(posf_v, [t3 + 1])
        dz = plsc.load_gather(posf_v, [s3 + 2]) - plsc.load_gather(posf_v, [t3 + 2])
        d2 = dx * dx + dy * dy + dz * dz
        zsv = plsc.load_gather(z_v, [srcv])
        d2_v[pl.ds(i * 16, 16)] = d2
        segv_v[pl.ds(i * 16, 16)] = dstv * MAXZ + zsv
        bv = lax.shift_right_logical(dstv, 9)
        for b in range(16):
            cv0 = cv0 + jnp.sum((bv == b).astype(jnp.int32)) * (lanes == b).astype(jnp.int32)
        for b in range(16, NB):
            cv1 = cv1 + jnp.sum((bv == b).astype(jnp.int32)) * (lanes == (b - 16)).astype(jnp.int32)
        return cv0, cv1

    zv16 = jnp.zeros((16,), jnp.int32)
    cv0, cv1 = lax.fori_loop(0, nv, vec_body, (zv16, zv16))
    pad8 = lambda v: lax.shift_left(lax.shift_right_logical(v + 7, 3), 3)
    cw_v[pl.ds(0, 16)] = pad8(cv0)
    cw_v[pl.ds(16, 16)] = jnp.where(lanes < NB - 16, pad8(cv1), 0)
    pltpu.sync_copy(cw_v, cnt_hbm.at[w])
    pltpu.sync_copy(d2_v, d2_hbm.at[pl.ds(start, CH)])
    pltpu.sync_copy(segv_v, seg_hbm.at[pl.ds(start, CH)])


@jax.jit
def _sc_prep(srcp, dstp, posf, z):
    return pl.kernel(
        _prep_body,
        out_type=(
            jax.ShapeDtypeStruct((EPAD,), jnp.float32),
            jax.ShapeDtypeStruct((EPAD,), jnp.int32),
            jax.ShapeDtypeStruct((NW, 32), jnp.int32),
        ),
        mesh=_MESH,
        scratch_types=[
            pltpu.VMEM((N * 3,), jnp.float32),
            pltpu.VMEM((N,), jnp.int32),
            pltpu.VMEM((CH,), jnp.int32),
            pltpu.VMEM((CH,), jnp.int32),
            pltpu.VMEM((CH,), jnp.float32),
            pltpu.VMEM((CH,), jnp.int32),
            pltpu.VMEM((32,), jnp.int32),
            pltpu.SMEM((32,), jnp.int32),
        ],
        compiler_params=pltpu.CompilerParams(needs_layout_passes=False),
    )(srcp, dstp, posf, z)


def _place_body(srce_hbm, dste_hbm, sege_hbm, cnts_hbm,
                srcs_hbm, segs_hbm, eids_hbm, cntp_hbm,
                src_v, dst_v, seg_v, bsrc_v, bseg_v, beid_v, cnts_v, cnt_loc,
                sm):
    c = lax.axis_index("c")
    s = lax.axis_index("s")
    w = s * 2 + c
    start = w * CH
    nume = jnp.where(w == NW - 1, E - (NW - 1) * CH, CH)
    nv = nume // 16
    lanes = lax.iota(jnp.int32, 16)

    pltpu.sync_copy(srce_hbm.at[pl.ds(start, CH)], src_v)
    pltpu.sync_copy(dste_hbm.at[pl.ds(start, CH)], dst_v)
    pltpu.sync_copy(sege_hbm.at[pl.ds(start, CH)], seg_v)
    pltpu.sync_copy(cnts_hbm, cnts_v.at[pl.ds(0, NW * 32)])

    loc = jnp.int32(0)
    gb = jnp.int32(0)
    for b in range(NB):
        sm[b] = loc
        sm[40 + b] = loc
        loc = loc + _cv(cnts_v, w * 32 + b)
        part = lax.fori_loop(0, w, lambda wp, a: a + _cv(cnts_v, wp * 32 + b), jnp.int32(0))
        tot = lax.fori_loop(0, NW, lambda wp, a: a + _cv(cnts_v, wp * 32 + b), jnp.int32(0))
        sm[20 + b] = gb + part
        gb = gb + tot

    def vec_body(i, _):
        srcv = src_v[pl.ds(i * 16, 16)]
        dstv = dst_v[pl.ds(i * 16, 16)]
        segv = seg_v[pl.ds(i * 16, 16)]
        eidv = start + i * 16 + lanes
        bv = lax.shift_right_logical(dstv, 9)
        for b in range(NB):
            m = bv == b
            off = sm[40 + b]
            plsc.store_compressed(bsrc_v.at[pl.ds(off, 16)], srcv, mask=m)
            plsc.store_compressed(bseg_v.at[pl.ds(off, 16)], segv, mask=m)
            plsc.store_compressed(beid_v.at[pl.ds(off, 16)], eidv, mask=m)
            sm[40 + b] = off + jnp.sum(m.astype(jnp.int32))
        return _

    lax.fori_loop(0, nv, vec_body, None)

    z16 = jnp.zeros((16,), jnp.int32)
    for b in range(NB):
        cur = sm[40 + b]
        end = sm[b] + _cv(cnts_v, w * 32 + b)
        m = lanes < (end - cur)
        plsc.store_compressed(bsrc_v.at[pl.ds(cur, 16)], z16, mask=m)
        plsc.store_compressed(bseg_v.at[pl.ds(cur, 16)],
                              jnp.full((16,), b * SEGB + SEGB, jnp.int32), mask=m)
        plsc.store_compressed(beid_v.at[pl.ds(cur, 16)], z16, mask=m)

    for b in range(NB):
        lo = sm[b]
        gbase = sm[20 + b]
        pcnt = _cv(cnts_v, w * 32 + b)
        n128 = lax.shift_right_logical(pcnt, 7)
        n8 = lax.shift_right_logical(pcnt - n128 * 128, 3)

        def big(j, _, lo=lo, gbase=gbase):
            go = pl.multiple_of(gbase + j * 128, 8)
            lo8 = pl.multiple_of(lo + j * 128, 8)
            pltpu.sync_copy(bsrc_v.at[pl.ds(lo8, 128)], srcs_hbm.at[pl.ds(go, 128)])
            pltpu.sync_copy(bseg_v.at[pl.ds(lo8, 128)], segs_hbm.at[pl.ds(go, 128)])
            pltpu.sync_copy(beid_v.at[pl.ds(lo8, 128)], eids_hbm.at[pl.ds(go, 128)])
            return _

        def small(j, _, lo=lo, gbase=gbase, n128=n128):
            o = n128 * 128 + j * 8
            go = pl.multiple_of(gbase + o, 8)
            lo8 = pl.multiple_of(lo + o, 8)
            pltpu.sync_copy(bsrc_v.at[pl.ds(lo8, 8)], srcs_hbm.at[pl.ds(go, 8)])
            pltpu.sync_copy(bseg_v.at[pl.ds(lo8, 8)], segs_hbm.at[pl.ds(go, 8)])
            pltpu.sync_copy(beid_v.at[pl.ds(lo8, 8)], eids_hbm.at[pl.ds(go, 8)])
            return _

        lax.fori_loop(0, n128, big, None)
        lax.fori_loop(0, n8, small, None)

    ones16 = jnp.ones((16,), jnp.int32)
    for b in range(NB):
        lo = sm[b]
        pcnt = _cv(cnts_v, w * 32 + b)

        def zero(j, _):
            cnt_loc[pl.ds(j * 16, 16)] = z16
            return _

        lax.fori_loop(0, 513, zero, None)

        def count(j, _, lo=lo, pcnt=pcnt):
            sv = bseg_v[pl.ds(lo + j * 16, 16)]
            idx = sv - b * SEGB
            m = lanes < (pcnt - j * 16)
            plsc.addupdate_scatter(cnt_loc, [idx], ones16, mask=m)
            return _

        lax.fori_loop(0, lax.shift_right_logical(pcnt + 15, 4), count, None)
        pltpu.sync_copy(cnt_loc, cntp_hbm.at[pl.ds(pl.multiple_of(w * (NB * 8208) + b * 8208, 8), 8208)])


@jax.jit
def _sc_place(srcp, dstp, segp, cnts):
    return pl.kernel(
        _place_body,
        out_type=(
            jax.ShapeDtypeStruct((EALLOC,), jnp.int32),
            jax.ShapeDtypeStruct((EALLOC,), jnp.int32),
            jax.ShapeDtypeStruct((EALLOC,), jnp.int32),
            jax.ShapeDtypeStruct((NW * NB * 8208,), jnp.int32),
        ),
        mesh=_MESH,
        scratch_types=[
            pltpu.VMEM((CH,), jnp.int32),
            pltpu.VMEM((CH,), jnp.int32),
            pltpu.VMEM((CH,), jnp.int32),
            pltpu.VMEM((5184,), jnp.int32),
            pltpu.VMEM((5184,), jnp.int32),
            pltpu.VMEM((5184,), jnp.int32),
            pltpu.VMEM((NW * 32 + 16,), jnp.int32),
            pltpu.VMEM((8208,), jnp.int32),
            pltpu.SMEM((64,), jnp.int32),
        ],
        compiler_params=pltpu.CompilerParams(needs_layout_passes=False),
    )(srcp, dstp, segp, cnts)


def _make_segsum(with_xc):
    def body(*refs):
        if with_xc:
            (wf_hbm, xc_hbm, srcs_hbm, segs_hbm, eids_hbm, cnts_hbm, y_hbm,
             acc_sh, zb_v, rows_v, xcr_v, eid_v, srci_v, idx_v, cnts_v, sm, sem) = refs
        else:
            (wf_hbm, srcs_hbm, segs_hbm, eids_hbm, cnts_hbm, y_hbm,
             acc_sh, zb_v, rows_v, eid_v, srci_v, idx_v, cnts_v, sm, sem) = refs
        c = lax.axis_index("c")
        s = lax.axis_index("s")
        lanes = lax.iota(jnp.int32, 16)
        pltpu.sync_copy(cnts_hbm, cnts_v.at[pl.ds(0, NW * 32)])
        gb = jnp.int32(0)
        for b in range(NB):
            sm[b] = gb
            gb = gb + lax.fori_loop(0, NW, lambda wp, a: a + _cv(cnts_v, wp * 32 + b), jnp.int32(0))
        sm[NB] = gb

        zf = jnp.zeros((16,), jnp.float32)

        def zrow(i, _):
            for cc in range(8):
                zb_v[i, pl.ds(cc * 16, 16)] = zf
            return _

        lax.fori_loop(0, 128, zrow, None)

        for j in range(NB // 2):
            b = 2 * j + c
            A = sm[b]
            Bnd = sm[b + 1]
            ln = Bnd - A
            for k in range(4):
                pltpu.sync_copy(zb_v, acc_sh.at[pl.ds(s * 512 + k * 128, 128)])
            plsc.subcore_barrier()
            nch = lax.shift_right_logical(ln + 127, 7)
            my = lax.shift_right_logical(nch - s + 15, 4)

            def chunk(jj, _, A=A, Bnd=Bnd, b=b):
                st = pl.multiple_of(A + (s + jj * 16) * 128, 8)
                pltpu.sync_copy(eids_hbm.at[pl.ds(st, 128)], eid_v)
                pltpu.sync_copy(segs_hbm.at[pl.ds(st, 128)], idx_v)
                if with_xc:
                    pltpu.sync_copy(srcs_hbm.at[pl.ds(st, 128)], srci_v)
                for t in range(8):
                    pos = st + t * 16 + lanes
                    ok = pos < Bnd
                    ev = eid_v[pl.ds(t * 16, 16)]
                    eid_v[pl.ds(t * 16, 16)] = jnp.where(ok, ev, 0)
                    sv = idx_v[pl.ds(t * 16, 16)]
                    idx_v[pl.ds(t * 16, 16)] = jnp.where(ok, sv - b * SEGB, SEGB)
                    if with_xc:
                        rv = srci_v[pl.ds(t * 16, 16)]
                        srci_v[pl.ds(t * 16, 16)] = jnp.where(ok, rv, 0)
                pltpu.async_copy(wf_hbm.at[eid_v], rows_v, sem).wait()
                if with_xc:
                    pltpu.async_copy(xc_hbm.at[srci_v], xcr_v, sem).wait()

                    def mul(r, _):
                        for cc in range(8):
                            rows_v[r, pl.ds(cc * 16, 16)] = (
                                rows_v[r, pl.ds(cc * 16, 16)] * xcr_v[r, pl.ds(cc * 16, 16)])
                        return _

                    lax.fori_loop(0, 128, mul, None, unroll=4)
                pltpu.sync_copy(rows_v, acc_sh.at[idx_v], add=True)
                return _

            lax.fori_loop(0, my, chunk, None)
            plsc.subcore_barrier()
            for k in range(4):
                pltpu.sync_copy(acc_sh.at[pl.ds(s * 512 + k * 128, 128)],
                                y_hbm.at[pl.ds(pl.multiple_of(b * SEGB + s * 512 + k * 128, 8), 128)])
            plsc.subcore_barrier()

    return body


def _segsum_call(with_xc):
    scratch = [
        pltpu.VMEM_SHARED((SEGB + 128, H), jnp.float32),
        pltpu.VMEM((128, H), jnp.float32),
        pltpu.VMEM((128, H), jnp.float32),
    ]
    if with_xc:
        scratch.append(pltpu.VMEM((128, H), jnp.float32))
    scratch += [
        pltpu.VMEM((128,), jnp.int32),
        pltpu.VMEM((128,), jnp.int32),
        pltpu.VMEM((128,), jnp.int32),
        pltpu.VMEM((NW * 32 + 16,), jnp.int32),
        pltpu.SMEM((64,), jnp.int32),
        pltpu.SemaphoreType.DMA,
    ]
    return functools.partial(
        pl.kernel,
        _make_segsum(with_xc),
        out_type=jax.ShapeDtypeStruct((NB * SEGB, H), jnp.float32),
        mesh=_MESH,
        scratch_types=scratch,
        compiler_params=pltpu.CompilerParams(needs_layout_passes=False),
    )()


@jax.jit
def _sc_segsum_plain(wf, srcs, segs, eids, cnts):
    return _segsum_call(False)(wf, srcs, segs, eids, cnts)


@jax.jit
def _sc_segsum_mul(wf, xc, srcs, segs, eids, cnts):
    return _segsum_call(True)(wf, xc, srcs, segs, eids, cnts)


def _edge_dense_body(d2_ref, means_ref, betas_ref, dproj_ref, dprojb_ref,
                     f1_ref, f1b_ref, f2_ref, f2b_ref,
                     wn_ref, wf0_ref, wf1_ref, wf2_ref):
    d2 = d2_ref[...].reshape(1024)
    d = jnp.sqrt(d2 + 1e-12)
    rbf = jnp.exp(-betas_ref[0][None, :] * (jnp.exp(-d)[:, None] - means_ref[0][None, :]) ** 2)
    Cc = (0.5 * (jnp.cos(d * (jnp.pi / CUT_HI)) + 1.0) * (d < CUT_HI))[:, None]
    wn_ref[...] = (jnp.dot(rbf, dproj_ref[...], preferred_element_type=jnp.float32)
                   + dprojb_ref[0][None, :]) * Cc
    for l, out_ref in enumerate((wf0_ref, wf1_ref, wf2_ref)):
        h1 = _silu(jnp.dot(rbf, f1_ref[l], preferred_element_type=jnp.float32) + f1b_ref[l][None, :])
        out_ref[...] = (jnp.dot(h1, f2_ref[l], preferred_element_type=jnp.float32)
                        + f2b_ref[l][None, :]) * Cc


@jax.jit
def _tc_edge_dense(d2p2, means, betas, dproj, dprojb, f1, f1b, f2, f2b):
    eo = jax.ShapeDtypeStruct((E, H), jnp.float32)
    full = lambda *shape: pl.BlockSpec(shape, lambda i: (0,) * len(shape))
    return pl.pallas_call(
        _edge_dense_body,
        grid=(157,),
        in_specs=[
            pl.BlockSpec((8, 128), lambda i: (i, 0)),
            full(1, NRBF), full(1, NRBF), full(NRBF, H), full(1, H),
            full(L, NRBF, H), full(L, H), full(L, H, H), full(L, H),
        ],
        out_specs=[pl.BlockSpec((1024, H), lambda i: (i, 0))] * 4,
        out_shape=(eo, eo, eo, eo),
    )(d2p2, means, betas, dproj, dprojb, f1, f1b, f2, f2b)


def _ne_body(z_ref, cntp_ref, wt_ref, emb_ref, ne_emb_ref, comb_ref, combb_ref,
             conv0_ref, x_ref, mask_ref, xc_ref):
    zb = z_ref[...]
    oh = (zb == lax.broadcasted_iota(jnp.int32, (1, MAXZ), 1)).astype(jnp.float32)
    xe = jnp.dot(oh, emb_ref[...], preferred_element_type=jnp.float32)
    wtb = wt_ref[...].reshape(128, MAXZ, H)
    agg = jnp.sum(wtb * ne_emb_ref[...][None, :, :], axis=1)
    x = (jnp.dot(xe, comb_ref[0:H, :], preferred_element_type=jnp.float32)
         + jnp.dot(agg, comb_ref[H:, :], preferred_element_type=jnp.float32)
         + combb_ref[0][None, :])
    x_ref[...] = x
    cnt = jnp.sum(cntp_ref[...], axis=0)[0]
    mask_ref[...] = (cnt > 0).astype(jnp.float32)
    xc_ref[...] = jnp.dot(x, conv0_ref[...], preferred_element_type=jnp.float32)


@jax.jit
def _tc_ne(z2, cntp4, wt, emb, ne_emb, comb, combb, conv0):
    full = lambda *shape: pl.BlockSpec(shape, lambda i: (0,) * len(shape))
    return pl.pallas_call(
        _ne_body,
        grid=(79,),
        in_specs=[
            pl.BlockSpec((128, 1), lambda i: (i, 0)),
            pl.BlockSpec((NW, 1, 128, MAXZ), lambda i: (0, i // 4, i % 4, 0)),
            pl.BlockSpec((2048, H), lambda i: (i, 0)),
            full(MAXZ, H), full(MAXZ, H), full(2 * H, H), full(1, H), full(H, H),
        ],
        out_specs=[
            pl.BlockSpec((128, H), lambda i: (i, 0)),
            pl.BlockSpec((128, MAXZ), lambda i: (i, 0)),
            pl.BlockSpec((128, H), lambda i: (i, 0)),
        ],
        out_shape=(
            jax.ShapeDtypeStruct((10112, H), jnp.float32),
            jax.ShapeDtypeStruct((10112, MAXZ), jnp.float32),
            jax.ShapeDtypeStruct((10112, H), jnp.float32),
        ),
    )(z2, cntp4, wt, emb, ne_emb, comb, combb, conv0)


def _make_attn_body(has_next):
    def body(*refs):
        if has_next:
            (y_ref, x_ref, mask_ref, qw, qb, kw, kb, vw, vb, ow, ob,
             blkw, blkb, convw, xo_ref, xco_ref) = refs
        else:
            (y_ref, x_ref, mask_ref, qw, qb, kw, kb, vw, vb, ow, ob,
             blkw, blkb, xo_ref) = refs
        yb = y_ref[...]
        q = jnp.dot(yb, qw[...], preferred_element_type=jnp.float32) + qb[0][None, :]
        k = jnp.dot(yb, kw[...], preferred_element_type=jnp.float32) + kb[0][None, :]
        v = jnp.dot(yb, vw[...], preferred_element_type=jnp.float32) + vb[0][None, :]
        maskb = mask_ref[...]
        q3 = q.reshape(128, MAXZ, H)
        k3 = k.reshape(128, MAXZ, H)
        v3 = v.reshape(128, MAXZ, H)
        colh = lax.broadcasted_iota(jnp.int32, (H, NH), 0) // DH
        rowh = lax.broadcasted_iota(jnp.int32, (H, NH), 1)
        bd = (colh == rowh).astype(jnp.float32)
        hid = lax.broadcasted_iota(jnp.int32, (NH, H), 0)
        cid = lax.broadcasted_iota(jnp.int32, (NH, H), 1)
        bdt = (cid // MAXZ == hid).astype(jnp.float32)
        S = jnp.zeros((128 * MAXZ, H), jnp.float32)
        for kk in range(MAXZ):
            t = (q3 * k3[:, kk, :][:, None, :]).reshape(128 * MAXZ, H)
            sk = jnp.dot(t, bd, preferred_element_type=jnp.float32)
            sel = (cid == hid * MAXZ + kk).astype(jnp.float32)
            S = S + jnp.dot(sk, sel, preferred_element_type=jnp.float32)
        maskcol = jnp.concatenate([maskb] * NH, axis=1)
        m2048 = jnp.broadcast_to(maskcol[:, None, :], (128, MAXZ, H)).reshape(128 * MAXZ, H)
        S = jnp.where(m2048 > 0, S, jnp.float32(-1e9))
        P3 = S.reshape(128 * MAXZ, NH, MAXZ)
        P3 = P3 - jnp.max(P3, axis=-1, keepdims=True)
        P3 = jnp.exp(P3)
        P3 = P3 / jnp.sum(P3, axis=-1, keepdims=True)
        out = jnp.zeros((128 * MAXZ, H), jnp.float32)
        for kk in range(MAXZ):
            pk = P3[:, :, kk]
            pk128 = jnp.dot(pk, bdt, preferred_element_type=jnp.float32)
            vkb = jnp.broadcast_to(v3[:, kk, :][:, None, :], (128, MAXZ, H)).reshape(128 * MAXZ, H)
            out = out + pk128 * vkb
        out = jnp.dot(out, ow[...], preferred_element_type=jnp.float32) + ob[0][None, :]
        out = out.reshape(128, MAXZ, H) * maskb[:, :, None]
        yn = _silu(jnp.sum(out, axis=1))
        xn = x_ref[...] + jnp.dot(yn, blkw[...], preferred_element_type=jnp.float32) + blkb[0][None, :]
        xo_ref[...] = xn
        if has_next:
            xco_ref[...] = jnp.dot(xn, convw[...], preferred_element_type=jnp.float32)
    return body


def _tc_attn(y, x, maskf, qw, qb, kw, kb, vw, vb, ow, ob, blkw, blkb, convw):
    has_next = convw is not None
    full = lambda *shape: pl.BlockSpec(shape, lambda i: (0,) * len(shape))
    w128 = full(H, H)
    b128 = full(1, H)
    in_specs = [
        pl.BlockSpec((2048, H), lambda i: (i, 0)),
        pl.BlockSpec((128, H), lambda i: (i, 0)),
        pl.BlockSpec((128, MAXZ), lambda i: (i, 0)),
        w128, b128, w128, b128, w128, b128, w128, b128, w128, b128,
    ]
    outs = [pl.BlockSpec((128, H), lambda i: (i, 0))]
    out_shape = [jax.ShapeDtypeStruct((10112, H), jnp.float32)]
    args = [y, x, maskf, qw, qb, kw, kb, vw, vb, ow, ob, blkw, blkb]
    if has_next:
        in_specs.append(w128)
        outs.append(pl.BlockSpec((128, H), lambda i: (i, 0)))
        out_shape.append(jax.ShapeDtypeStruct((10112, H), jnp.float32))
        args.append(convw)
    return pl.pallas_call(
        _make_attn_body(has_next),
        grid=(79,),
        in_specs=in_specs,
        out_specs=outs,
        out_shape=tuple(out_shape),
        compiler_params=pltpu.CompilerParams(vmem_limit_bytes=100 * 1024 * 1024),
    )(*args)


def kernel(z, pos, edge_index, emb, ne_emb, ne_dproj_w, ne_dproj_b, ne_comb_w, ne_comb_b, rbf_means, rbf_betas, conv_lin1_w, filt1_w, filt1_b, filt2_w, filt2_b, q_w, q_b, k_w, k_b, v_w, v_b, o_w, o_b, blk_w, blk_b):
    ei_p = jnp.pad(edge_index.astype(jnp.int32), ((0, 0), (0, EPAD - E)))
    posf = pos.reshape(-1)
    zi = z.astype(jnp.int32)
    d2p, segp, cnts = _sc_prep(ei_p[0], ei_p[1], posf, zi)
    cnts1d = cnts.reshape(-1)
    srcs, segs, eids, cntp = _sc_place(ei_p[0], ei_p[1], segp, cnts1d)
    wn, wf0, wf1, wf2 = _tc_edge_dense(
        d2p.reshape(1252, 128), rbf_means.reshape(1, NRBF), rbf_betas.reshape(1, NRBF),
        ne_dproj_w, ne_dproj_b.reshape(1, H), filt1_w, filt1_b, filt2_w, filt2_b)
    wt = _sc_segsum_plain(wn, srcs, segs, eids, cnts1d)
    xp, maskp, xc = _tc_ne(
        zi.reshape(N, 1), cntp.reshape(NW, NB, 513, MAXZ), wt, emb, ne_emb,
        ne_comb_w, ne_comb_b.reshape(1, H), conv_lin1_w[0])
    wfs = (wf0, wf1, wf2)
    NP = 10112
    present = maskp > 0
    for l in range(L):
        yl = _sc_segsum_mul(wfs[l], xc, srcs, segs, eids, cnts1d)
        y = yl[:NP * MAXZ].reshape(NP, MAXZ, H)
        q = (y @ q_w[l] + q_b[l]).reshape(NP, MAXZ, NH, DH)
        k = (y @ k_w[l] + k_b[l]).reshape(NP, MAXZ, NH, DH)
        v = (y @ v_w[l] + v_b[l]).reshape(NP, MAXZ, NH, DH)
        attn = jnp.einsum('nmhd,nkhd->nhmk', q, k)
        attn = jnp.where(present[:, None, None, :], attn, jnp.float32(-1e9))
        p = jax.nn.softmax(attn, axis=-1)
        out = jnp.einsum('nhmk,nkhd->nmhd', p, v).reshape(NP, MAXZ, H)
        out = (out @ o_w[l] + o_b[l]) * maskp[:, :, None]
        yn = _silu(jnp.sum(out, axis=1))
        xp = xp + yn @ blk_w[l] + blk_b[l]
        if l < L - 1:
            xc = xp @ conv_lin1_w[l + 1]
    return xp[:N]
